# baseline (speedup 1.0000x reference)
import jax
import jax.numpy as jnp
from jax.experimental import pallas as pl
from jax.experimental.pallas import tpu as pltpu

_BM = 4992
_STEPS = 4
_NS = 5
_QUARTER = _BM * _STEPS
_TAIL = 100000 - _NS * _QUARTER
_TIDX = (_NS * _QUARTER) // _TAIL


def _mlp_block(*refs):
    xs = refs[:_NS]
    xc_ref, w1_ref, b1_ref, w2t_ref, b2_ref, o_ref = refs[_NS:_NS + 6]
    obufs = refs[_NS + 6:2 * _NS + 6]
    obufc, osem = refs[2 * _NS + 6:]

    i = pl.program_id(0)
    slot = jax.lax.rem(i, 2)

    def stripe_copy(s, step, sl):
        return pltpu.make_async_copy(
            obufs[s].at[sl],
            o_ref.at[:, pl.ds(s * _QUARTER + step * _BM, _BM)],
            osem.at[s, sl])

    @pl.when(i >= 2)
    def _():
        for s in range(_NS):
            stripe_copy(s, i - 2, slot).wait()

    def tile(x):
        h = jnp.dot(x, w1_ref[...], preferred_element_type=jnp.float32)
        h = jnp.maximum(h + b1_ref[...], 0.0)
        ot = jax.lax.dot_general(
            w2t_ref[...].astype(jnp.bfloat16), h.astype(jnp.bfloat16),
            (((1,), (1,)), ((), ())), preferred_element_type=jnp.float32)
        return ot + b2_ref[...]

    for s in range(_NS):
        obufs[s][slot] = tile(xs[s][...])
        stripe_copy(s, i, slot).start()

    @pl.when(i == _STEPS - 1)
    def _():
        obufc[...] = tile(xc_ref[...])
        tail_copy = pltpu.make_async_copy(
            obufc, o_ref.at[:, pl.ds(_NS * _QUARTER, _TAIL)], osem.at[_NS, 0])
        tail_copy.start()
        for s in range(_NS):
            stripe_copy(s, i - 1, 1 - slot).wait()
        for s in range(_NS):
            stripe_copy(s, i, slot).wait()
        pltpu.make_async_copy(
            obufc, o_ref.at[:, pl.ds(_NS * _QUARTER, _TAIL)],
            osem.at[_NS, 0]).wait()


def kernel(features, W1, b1, W2, b2):
    m, d = features.shape
    d_hid = W1.shape[1]
    n_cls = W2.shape[1]
    stream_specs = [
        pl.BlockSpec((_BM, d), lambda i, s=s: (s * _STEPS + i, 0))
        for s in range(_NS)
    ]
    out_t = pl.pallas_call(
        _mlp_block,
        grid=(_STEPS,),
        in_specs=stream_specs + [
            pl.BlockSpec((_TAIL, d), lambda i: (_TIDX, 0)),
            pl.BlockSpec((d, d_hid), lambda i: (0, 0)),
            pl.BlockSpec((1, d_hid), lambda i: (0, 0)),
            pl.BlockSpec((n_cls, d_hid), lambda i: (0, 0)),
            pl.BlockSpec((n_cls, 1), lambda i: (0, 0)),
        ],
        out_specs=pl.BlockSpec(memory_space=pltpu.MemorySpace.HBM),
        out_shape=jax.ShapeDtypeStruct((n_cls, m), jnp.float32),
        scratch_shapes=(
            [pltpu.VMEM((2, n_cls, _BM), jnp.float32) for _ in range(_NS)]
            + [
                pltpu.VMEM((n_cls, _TAIL), jnp.float32),
                pltpu.SemaphoreType.DMA((_NS + 1, 2)),
            ]
        ),
        compiler_params=pltpu.CompilerParams(
            dimension_semantics=("arbitrary",),
        ),
    )(*([features] * _NS), features, W1, b1.reshape(1, -1), W2.T,
      b2.reshape(-1, 1))
    return out_t.T


# speedup vs baseline: 1.0322x; 1.0322x over previous
import jax
import jax.numpy as jnp
from jax.experimental import pallas as pl
from jax.experimental.pallas import tpu as pltpu

_BM = 4992
_STEPS = 5
_NS = 4
_QUARTER = _BM * _STEPS
_TAIL = 100000 - _NS * _QUARTER
_TIDX = (_NS * _QUARTER) // _TAIL


def _mlp_block(*refs):
    xs = refs[:_NS]
    xc_ref, w1_ref, b1_ref, w2t_ref, b2_ref, o_ref = refs[_NS:_NS + 6]
    obufs = refs[_NS + 6:2 * _NS + 6]
    obufc, osem = refs[2 * _NS + 6:]

    i = pl.program_id(0)
    slot = jax.lax.rem(i, 2)

    def stripe_copy(s, step, sl):
        return pltpu.make_async_copy(
            obufs[s].at[sl],
            o_ref.at[:, pl.ds(s * _QUARTER + step * _BM, _BM)],
            osem.at[s, sl])

    @pl.when(i >= 2)
    def _():
        for s in range(_NS):
            stripe_copy(s, i - 2, slot).wait()

    def tile(x):
        h = jnp.dot(x, w1_ref[...], preferred_element_type=jnp.float32)
        h = jnp.maximum(h + b1_ref[...], 0.0)
        ot = jax.lax.dot_general(
            w2t_ref[...].astype(jnp.bfloat16), h.astype(jnp.bfloat16),
            (((1,), (1,)), ((), ())), preferred_element_type=jnp.float32)
        return ot + b2_ref[...]

    for s in range(_NS):
        obufs[s][slot] = tile(xs[s][...])
        stripe_copy(s, i, slot).start()

    @pl.when(i == _STEPS - 1)
    def _():
        obufc[...] = tile(xc_ref[...])
        tail_copy = pltpu.make_async_copy(
            obufc, o_ref.at[:, pl.ds(_NS * _QUARTER, _TAIL)], osem.at[_NS, 0])
        tail_copy.start()
        for s in range(_NS):
            stripe_copy(s, i - 1, 1 - slot).wait()
        for s in range(_NS):
            stripe_copy(s, i, slot).wait()
        pltpu.make_async_copy(
            obufc, o_ref.at[:, pl.ds(_NS * _QUARTER, _TAIL)],
            osem.at[_NS, 0]).wait()


def kernel(features, W1, b1, W2, b2):
    m, d = features.shape
    d_hid = W1.shape[1]
    n_cls = W2.shape[1]
    stream_specs = [
        pl.BlockSpec((_BM, d), lambda i, s=s: (s * _STEPS + i, 0))
        for s in range(_NS)
    ]
    out_t = pl.pallas_call(
        _mlp_block,
        grid=(_STEPS,),
        in_specs=stream_specs + [
            pl.BlockSpec((_TAIL, d), lambda i: (_TIDX, 0)),
            pl.BlockSpec((d, d_hid), lambda i: (0, 0)),
            pl.BlockSpec((1, d_hid), lambda i: (0, 0)),
            pl.BlockSpec((n_cls, d_hid), lambda i: (0, 0)),
            pl.BlockSpec((n_cls, 1), lambda i: (0, 0)),
        ],
        out_specs=pl.BlockSpec(memory_space=pltpu.MemorySpace.HBM),
        out_shape=jax.ShapeDtypeStruct((n_cls, m), jnp.float32),
        scratch_shapes=(
            [pltpu.VMEM((2, n_cls, _BM), jnp.float32) for _ in range(_NS)]
            + [
                pltpu.VMEM((n_cls, _TAIL), jnp.float32),
                pltpu.SemaphoreType.DMA((_NS + 1, 2)),
            ]
        ),
        compiler_params=pltpu.CompilerParams(
            dimension_semantics=("arbitrary",),
        ),
    )(*([features] * _NS), features, W1, b1.reshape(1, -1), W2.T,
      b2.reshape(-1, 1))
    return out_t.T
